# Initial kernel scaffold; baseline (speedup 1.0000x reference)
import jax
import jax.numpy as jnp
from jax.experimental import pallas as pl


def kernel(x):
    raise NotImplementedError("write your pallas kernel here")


# speedup vs baseline: 1.0086x; 1.0086x over previous
import dataclasses

import jax
import jax.numpy as jnp
from jax import lax
from jax.experimental import pallas as pl
from jax.experimental.pallas import tpu as pltpu
from jax.experimental.pallas import tpu_sc as plsc

N_CLASSES = 1000
BATCH = 16384
NUM_WORKERS = 32
ROWS_PER_WORKER = BATCH // NUM_WORKERS
BLOCK_ROWS = 32
NUM_BLOCKS = ROWS_PER_WORKER // BLOCK_ROWS
BLOCK_ELEMS = BLOCK_ROWS * N_CLASSES
LANES = 16


def _onehot_flat(x):
    mesh = plsc.VectorSubcoreMesh(core_axis_name="c", subcore_axis_name="s")
    cp = pltpu.CompilerParams()
    if "needs_layout_passes" in pltpu.CompilerParams.__dataclass_fields__:
        cp = dataclasses.replace(cp, needs_layout_passes=False)

    @pl.kernel(
        compiler_params=cp,
        out_type=jax.ShapeDtypeStruct((BATCH * N_CLASSES,), jnp.float32),
        mesh=mesh,
        scratch_types=[
            pltpu.VMEM((ROWS_PER_WORKER,), jnp.int32),
            pltpu.VMEM((BLOCK_ELEMS,), jnp.float32),
        ],
    )
    def body(x_hbm, out_hbm, idx_v, buf):
        wid = lax.axis_index("s") * 2 + lax.axis_index("c")
        row0 = wid * ROWS_PER_WORKER

        pltpu.sync_copy(x_hbm.at[pl.ds(row0, ROWS_PER_WORKER)], idx_v)

        zeros16 = jnp.zeros((LANES,), jnp.float32)
        ones16 = jnp.ones((LANES,), jnp.float32)
        lane_iota = lax.iota(jnp.int32, LANES)

        @pl.loop(0, BLOCK_ELEMS, step=LANES)
        def _(i):
            buf[pl.ds(i, LANES)] = zeros16

        out_base = row0 * N_CLASSES

        @pl.loop(0, NUM_BLOCKS)
        def _(b):
            flats = []
            for j in range(BLOCK_ROWS // LANES):
                cols = idx_v[pl.ds(b * BLOCK_ROWS + j * LANES, LANES)]
                flat = (lane_iota + j * LANES) * N_CLASSES + cols
                flats.append(flat)
                plsc.store_scatter(buf, [flat], ones16)
            pltpu.sync_copy(buf, out_hbm.at[pl.ds(out_base + b * BLOCK_ELEMS,
                                                  BLOCK_ELEMS)])
            for flat in flats:
                plsc.store_scatter(buf, [flat], zeros16)

    return body(x)


def kernel(x):
    out = _onehot_flat(x.astype(jnp.int32))
    return out.reshape(BATCH, N_CLASSES)
